# baseline (speedup 1.0000x reference)
import functools
import jax
import jax.numpy as jnp
from jax import lax
from jax.experimental import pallas as pl
from jax.experimental.pallas import tpu as pltpu, tpu_sc as plsc

SEQ = 8192
BATCH = 4
D = 1024
NW = 32
ROWS_PER_W = SEQ // NW
CHUNK = 8
NCHUNK = ROWS_PER_W // CHUNK
NBUF = 2
GROUPS = CHUNK * (D // 16)


def _sc_body(x_hbm, pos_hbm, out_hbm, xbufs, pbufs, insems, outsems):
    c = lax.axis_index("c")
    s = lax.axis_index("s")
    wid = s * 2 + c
    base = wid * ROWS_PER_W

    def start_in(chunk, slot):
        row = base + chunk * CHUNK
        pltpu.async_copy(x_hbm.at[pl.ds(row, CHUNK)], xbufs.at[slot], insems.at[slot])
        pltpu.async_copy(pos_hbm.at[pl.ds(row, CHUNK)], pbufs.at[slot], insems.at[slot])

    def compute(slot):
        xb = xbufs.at[slot]
        pb = pbufs.at[slot]

        def body(g, carry):
            sr = g // (D // 16)
            j = g % (D // 16)
            pvec = pb[sr, pl.ds(j * 16, 16)]
            for b in range(BATCH):
                xb[sr, b, pl.ds(j * 16, 16)] += pvec
            return carry

        lax.fori_loop(0, GROUPS, body, 0, unroll=2)

    def start_out(chunk, slot):
        row = base + chunk * CHUNK
        pltpu.async_copy(xbufs.at[slot], out_hbm.at[pl.ds(row, CHUNK)], outsems.at[slot])

    for b in range(NBUF):
        start_in(b, b)

    def loop(chunk, carry):
        slot = lax.rem(chunk, NBUF)
        pltpu.make_async_copy(x_hbm.at[pl.ds(0, CHUNK)], xbufs.at[slot], insems.at[slot]).wait()
        pltpu.make_async_copy(pos_hbm.at[pl.ds(0, CHUNK)], pbufs.at[slot], insems.at[slot]).wait()
        compute(slot)
        start_out(chunk, slot)
        @pl.when(chunk + NBUF < NCHUNK)
        def refill():
            pltpu.make_async_copy(xbufs.at[slot], out_hbm.at[pl.ds(0, CHUNK)], outsems.at[slot]).wait()
            start_in(chunk + NBUF, slot)
        return carry

    lax.fori_loop(0, NCHUNK, loop, 0)
    for b in range(NBUF):
        last = NCHUNK - NBUF + b
        slot = last % NBUF
        pltpu.make_async_copy(xbufs.at[slot], out_hbm.at[pl.ds(0, CHUNK)], outsems.at[slot]).wait()


def kernel(x, pos_embedding):
    mesh = plsc.VectorSubcoreMesh(core_axis_name="c", subcore_axis_name="s")
    k = functools.partial(
        pl.kernel,
        mesh=mesh,
        out_type=jax.ShapeDtypeStruct((SEQ, BATCH, D), jnp.float32),
        scratch_types=[
            pltpu.VMEM((NBUF, CHUNK, BATCH, D), jnp.float32),
            pltpu.VMEM((NBUF, CHUNK, D), jnp.float32),
            pltpu.SemaphoreType.DMA((NBUF,)),
            pltpu.SemaphoreType.DMA((NBUF,)),
        ],
    )(_sc_body)
    return k(x, pos_embedding[:SEQ])


# speedup vs baseline: 1.7000x; 1.7000x over previous
import jax
import jax.numpy as jnp
from jax.experimental import pallas as pl


def _add_kernel(x_ref, pos_ref, o_ref):
    o_ref[...] = x_ref[...] + pos_ref[...][:, None, :]


def kernel(x, pos_embedding):
    seq, batch, d = x.shape
    tile = 512
    dtile = 512
    grid = (seq // tile, d // dtile)
    return pl.pallas_call(
        _add_kernel,
        grid=grid,
        in_specs=[
            pl.BlockSpec((tile, batch, dtile), lambda i, j: (i, 0, j)),
            pl.BlockSpec((tile, dtile), lambda i, j: (i, j)),
        ],
        out_specs=pl.BlockSpec((tile, batch, dtile), lambda i, j: (i, 0, j)),
        out_shape=jax.ShapeDtypeStruct((seq, batch, d), x.dtype),
    )(x, pos_embedding[:seq])
